# baseline (device time: 34705 ns/iter reference)
import jax
import jax.numpy as jnp
from jax import lax
from jax.experimental import pallas as pl
from jax.experimental.pallas import tpu as pltpu

N_DEV = 16


def kernel(x, w_mat):
    k_total, n_cols = x.shape
    _, n_out = w_mat.shape
    m_per = k_total // N_DEV

    def body(x_ref, w_hbm, out_ref, x_bf, x_row, w_f32, w_bf, send_sems,
             recv_sems, w_dma_sem):
        my = lax.axis_index("i")

        w_dma = pltpu.make_async_copy(w_hbm, w_f32, w_dma_sem)
        w_dma.start()

        def send_descriptor(k):
            dst = (my + k) % N_DEV
            return pltpu.make_async_remote_copy(
                src_ref=x_bf.at[pl.ds(dst * m_per, m_per), :],
                dst_ref=x_row.at[:, pl.ds(my * n_cols, n_cols)],
                send_sem=send_sems.at[k],
                recv_sem=recv_sems.at[my],
                device_id=(dst,),
                device_id_type=pl.DeviceIdType.MESH,
            )

        x_bf[:, :] = x_ref[:, :].astype(jnp.bfloat16)
        x_row[:, pl.ds(my * n_cols, n_cols)] = x_bf[pl.ds(my * m_per, m_per), :]

        barrier_sem = pltpu.get_barrier_semaphore()
        for k in range(1, N_DEV):
            peer = (my + k) % N_DEV
            pl.semaphore_signal(
                barrier_sem, inc=1,
                device_id=(peer,), device_id_type=pl.DeviceIdType.MESH,
            )
        pl.semaphore_wait(barrier_sem, N_DEV - 1)

        for k in range(1, N_DEV):
            send_descriptor(k).start()

        w_dma.wait()
        w_bf[:, :] = w_f32[:, :].astype(jnp.bfloat16)

        def kblock_dot(j):
            return jnp.dot(
                x_row[:, pl.ds(j * n_cols, n_cols)],
                w_bf[pl.ds(j * n_cols, n_cols), :],
                preferred_element_type=jnp.float32,
            )

        out_ref[:, :] = kblock_dot(my)
        for k in range(1, N_DEV):
            j = (my - k) % N_DEV
            recv = pltpu.make_async_remote_copy(
                src_ref=x_bf.at[pl.ds(0, m_per), :],
                dst_ref=x_row.at[:, pl.ds(j * n_cols, n_cols)],
                send_sem=send_sems.at[0],
                recv_sem=recv_sems.at[j],
                device_id=(my,),
                device_id_type=pl.DeviceIdType.MESH,
            )
            recv.wait_recv()
            out_ref[:, :] += kblock_dot(j)

        y = out_ref[:, :]
        out_ref[:, :] = y * jax.nn.sigmoid(y)

        for k in range(1, N_DEV):
            send_descriptor(k).wait_send()

    return pl.pallas_call(
        body,
        out_shape=jax.ShapeDtypeStruct((m_per, n_out), jnp.float32),
        in_specs=[
            pl.BlockSpec(memory_space=pltpu.VMEM),
            pl.BlockSpec(memory_space=pltpu.MemorySpace.HBM),
        ],
        out_specs=pl.BlockSpec(memory_space=pltpu.VMEM),
        scratch_shapes=[
            pltpu.VMEM((k_total, n_cols), jnp.bfloat16),
            pltpu.VMEM((m_per, k_total), jnp.bfloat16),
            pltpu.VMEM((k_total, n_out), jnp.float32),
            pltpu.VMEM((k_total, n_out), jnp.bfloat16),
            pltpu.SemaphoreType.DMA((N_DEV,)),
            pltpu.SemaphoreType.DMA((N_DEV,)),
            pltpu.SemaphoreType.DMA,
        ],
        compiler_params=pltpu.CompilerParams(
            collective_id=0,
            vmem_limit_bytes=96 * 1024 * 1024,
        ),
    )(x, w_mat)
